# baseline (speedup 1.0000x reference)
import jax
import jax.numpy as jnp
from jax import lax
from jax.experimental import pallas as pl
from jax.experimental.pallas import tpu as pltpu

TM = 256
NBUF = 8


def _sage_kern(adj_hbm, xt_ref, w_ref, ot_ref, buf, sem):
    n_rows, S = adj_hbm.shape
    num_chunks = n_rows // TM
    blocks_per_batch = S // TM
    w = w_ref[...]

    def chunk_copy(i, slot):
        return pltpu.make_async_copy(
            adj_hbm.at[pl.ds(i * TM, TM), :],
            buf.at[slot],
            sem.at[slot],
        )

    for k in range(NBUF - 1):
        chunk_copy(k, k).start()

    def body(i, _):
        slot = lax.rem(i, NBUF)
        chunk_copy(i, slot).wait()
        nxt = i + NBUF - 1
        @pl.when(nxt < num_chunks)
        def _start_next():
            chunk_copy(nxt, lax.rem(nxt, NBUF)).start()
        b = lax.div(i, blocks_per_batch)
        xbt = xt_ref[:, pl.ds(b * S, S)]
        a = buf[slot]
        tmpt = lax.dot_general(
            xbt, a, (((1,), (1,)), ((), ())),
            preferred_element_type=jnp.float32)
        rest = tmpt + xt_ref[:, pl.ds(i * TM, TM)]
        ot_ref[:, pl.ds(i * TM, TM)] = lax.dot_general(
            w, rest, (((0,), (0,)), ((), ())),
            preferred_element_type=jnp.float32)
        return 0

    lax.fori_loop(0, num_chunks, body, 0)


def kernel(x, adj_t, W):
    B, S, _ = adj_t.shape
    N, IN = x.shape
    OUT = W.shape[1]
    adj2d = adj_t.reshape(N, S)
    xt = x.T

    outt = pl.pallas_call(
        _sage_kern,
        in_specs=[
            pl.BlockSpec(memory_space=pltpu.MemorySpace.HBM),
            pl.BlockSpec(memory_space=pltpu.MemorySpace.VMEM),
            pl.BlockSpec(memory_space=pltpu.MemorySpace.VMEM),
        ],
        out_specs=pl.BlockSpec(memory_space=pltpu.MemorySpace.VMEM),
        out_shape=jax.ShapeDtypeStruct((OUT, N), jnp.float32),
        scratch_shapes=[
            pltpu.VMEM((NBUF, TM, S), jnp.float32),
            pltpu.SemaphoreType.DMA((NBUF,)),
        ],
    )(adj2d, xt, W)
    return outt.T


# speedup vs baseline: 1.0114x; 1.0114x over previous
import jax
import jax.numpy as jnp
from jax import lax
from jax.experimental import pallas as pl
from jax.experimental.pallas import tpu as pltpu

TM = 512
NBUF = 4


def _sage_kern(adj_hbm, xt_ref, w_ref, ot_hbm, buf, sem, ob, osem):
    n_rows, S = adj_hbm.shape
    num_chunks = n_rows // TM
    blocks_per_batch = S // TM
    w = w_ref[...]

    def chunk_copy(i, slot):
        return pltpu.make_async_copy(
            adj_hbm.at[pl.ds(i * TM, TM), :],
            buf.at[slot],
            sem.at[slot],
        )

    def out_copy(i, slot):
        return pltpu.make_async_copy(
            ob.at[slot],
            ot_hbm.at[:, pl.ds(i * TM, TM)],
            osem.at[slot],
        )

    for k in range(NBUF - 1):
        chunk_copy(k, k).start()

    def body(i, _):
        slot = lax.rem(i, NBUF)
        chunk_copy(i, slot).wait()
        nxt = i + NBUF - 1
        @pl.when(nxt < num_chunks)
        def _start_next():
            chunk_copy(nxt, lax.rem(nxt, NBUF)).start()
        b = lax.div(i, blocks_per_batch)
        xbt = xt_ref[:, pl.ds(b * S, S)]
        a = buf[slot]
        tmpt = lax.dot_general(
            xbt, a, (((1,), (1,)), ((), ())),
            preferred_element_type=jnp.float32)
        rest = tmpt + xt_ref[:, pl.ds(i * TM, TM)]
        oslot = lax.rem(i, 2)
        @pl.when(i >= 2)
        def _drain_out():
            out_copy(i - 2, oslot).wait()
        ob[oslot] = lax.dot_general(
            w, rest, (((0,), (0,)), ((), ())),
            preferred_element_type=jnp.float32)
        out_copy(i, oslot).start()
        return 0

    lax.fori_loop(0, num_chunks, body, 0)
    out_copy(num_chunks - 2, lax.rem(num_chunks - 2, 2)).wait()
    out_copy(num_chunks - 1, lax.rem(num_chunks - 1, 2)).wait()


def kernel(x, adj_t, W):
    B, S, _ = adj_t.shape
    N, IN = x.shape
    OUT = W.shape[1]
    adj2d = adj_t.reshape(N, S)
    xt = x.T

    outt = pl.pallas_call(
        _sage_kern,
        in_specs=[
            pl.BlockSpec(memory_space=pltpu.MemorySpace.HBM),
            pl.BlockSpec(memory_space=pltpu.MemorySpace.VMEM),
            pl.BlockSpec(memory_space=pltpu.MemorySpace.VMEM),
        ],
        out_specs=pl.BlockSpec(memory_space=pltpu.MemorySpace.HBM),
        out_shape=jax.ShapeDtypeStruct((OUT, N), jnp.float32),
        scratch_shapes=[
            pltpu.VMEM((NBUF, TM, S), jnp.float32),
            pltpu.SemaphoreType.DMA((NBUF,)),
            pltpu.VMEM((2, OUT, TM), jnp.float32),
            pltpu.SemaphoreType.DMA((2,)),
        ],
    )(adj2d, xt, W)
    return outt.T


# speedup vs baseline: 1.0137x; 1.0022x over previous
import jax
import jax.numpy as jnp
from jax import lax
from jax.experimental import pallas as pl
from jax.experimental.pallas import tpu as pltpu

TM = 512
NBUF = 4


def _sage_kern(adj_hbm, xt_ref, w_ref, ot_ref, buf, sem):
    n_rows, S = adj_hbm.shape
    num_chunks = n_rows // TM
    blocks_per_batch = S // TM
    w = w_ref[...]

    def chunk_copy(i, slot):
        return pltpu.make_async_copy(
            adj_hbm.at[pl.ds(i * TM, TM), :],
            buf.at[slot],
            sem.at[slot],
        )

    for k in range(NBUF - 1):
        chunk_copy(k, k).start()

    def body(i, _):
        slot = lax.rem(i, NBUF)
        chunk_copy(i, slot).wait()
        nxt = i + NBUF - 1
        @pl.when(nxt < num_chunks)
        def _start_next():
            chunk_copy(nxt, lax.rem(nxt, NBUF)).start()
        b = lax.div(i, blocks_per_batch)
        xbt = xt_ref[:, pl.ds(b * S, S)]
        a = buf[slot]
        tmpt = lax.dot_general(
            xbt, a, (((1,), (1,)), ((), ())),
            preferred_element_type=jnp.float32)
        rest = tmpt + xt_ref[:, pl.ds(i * TM, TM)]
        ot_ref[:, pl.ds(i * TM, TM)] = lax.dot_general(
            w, rest, (((0,), (0,)), ((), ())),
            preferred_element_type=jnp.float32)
        return 0

    lax.fori_loop(0, num_chunks, body, 0)


def kernel(x, adj_t, W):
    B, S, _ = adj_t.shape
    N, IN = x.shape
    OUT = W.shape[1]
    adj2d = adj_t.reshape(N, S)
    xt = x.T

    outt = pl.pallas_call(
        _sage_kern,
        in_specs=[
            pl.BlockSpec(memory_space=pltpu.MemorySpace.HBM),
            pl.BlockSpec(memory_space=pltpu.MemorySpace.VMEM),
            pl.BlockSpec(memory_space=pltpu.MemorySpace.VMEM),
        ],
        out_specs=pl.BlockSpec(memory_space=pltpu.MemorySpace.VMEM),
        out_shape=jax.ShapeDtypeStruct((OUT, N), jnp.float32),
        scratch_shapes=[
            pltpu.VMEM((NBUF, TM, S), jnp.float32),
            pltpu.SemaphoreType.DMA((NBUF,)),
        ],
    )(adj2d, xt, W)
    return outt.T
